# baseline (speedup 1.0000x reference)
import functools

import jax
import jax.numpy as jnp
from jax import lax
from jax.experimental import pallas as pl
from jax.experimental.pallas import tpu as pltpu
from jax.experimental.pallas import tpu_sc as plsc

NODE_D = 128
HID = 64
EDGE_D = 16

_NC = 2
_NS = 16
_NW = _NC * _NS


def _elu(x):
    return jnp.where(x > 0, x, jnp.exp(x) - 1.0)


def _softplus(x):
    return jnp.maximum(x, 0.0) + jnp.log(1.0 + jnp.exp(-jnp.abs(x)))


def _precompute_body(nf_ref, wst_ref, wrt_ref, ps_ref, pr_ref):
    nf = nf_ref[...]
    ps_ref[...] = jnp.dot(nf, wst_ref[...], preferred_element_type=jnp.float32)
    pr_ref[...] = jnp.dot(nf, wrt_ref[...], preferred_element_type=jnp.float32)


def _mlp_body(g2_ref, eflo_ref, efhi_ref, w1et_ref, b1_ref, w2t_ref, b2_ref,
              w3_ref, b3_ref, outlo_ref, outhi_ref):
    g2 = g2_ref[...]
    ef = jnp.concatenate([eflo_ref[...], efhi_ref[...]], axis=0)
    g = jnp.concatenate([g2[:, :HID], g2[:, HID:]], axis=0)
    a = jnp.dot(ef, w1et_ref[...], preferred_element_type=jnp.float32)
    h1 = _elu(g + a + b1_ref[...])
    h2 = _elu(jnp.dot(h1, w2t_ref[...], preferred_element_type=jnp.float32)
              + b2_ref[...])
    z = lax.dot_general(w3_ref[...], h2, (((1,), (1,)), ((), ())),
                        preferred_element_type=jnp.float32)
    sp = _softplus(z + b3_ref[...])
    half = sp.shape[1] // 2
    outlo_ref[0] = sp[:, :half]
    outhi_ref[0] = sp[:, half:]


def _make_gather(n_edges, cpairs):
    half = n_edges // 2
    ppw = half // _NW
    nchunk = ppw // cpairs
    mesh = plsc.VectorSubcoreMesh(core_axis_name="c", subcore_axis_name="s")

    @functools.partial(
        pl.kernel,
        mesh=mesh,
        compiler_params=pltpu.CompilerParams(use_tc_tiling_on_sc=False),
        out_type=jax.ShapeDtypeStruct((half, 2 * HID), jnp.float32),
        scratch_types=[
            pltpu.VMEM((cpairs,), jnp.int32),
            pltpu.VMEM((cpairs,), jnp.int32),
            pltpu.VMEM((cpairs,), jnp.int32),
            pltpu.VMEM((cpairs,), jnp.int32),
            pltpu.VMEM((cpairs, HID), jnp.float32),
            pltpu.VMEM((cpairs, HID), jnp.float32),
            pltpu.VMEM((cpairs, HID), jnp.float32),
            pltpu.VMEM((cpairs, HID), jnp.float32),
            pltpu.VMEM((cpairs, 2 * HID), jnp.float32),
            pltpu.SemaphoreType.DMA,
        ],
    )
    def _gather(ps_hbm, pr_hbm, s_hbm, r_hbm, out_hbm, islo, irlo, ishi, irhi,
                bslo, brlo, bshi, brhi, bout, sem):
        wid = lax.axis_index("s") * _NC + lax.axis_index("c")
        base = wid * ppw

        def chunk_body(k, carry):
            off = base + k * cpairs
            pltpu.sync_copy(s_hbm.at[pl.ds(off, cpairs)], islo)
            pltpu.sync_copy(r_hbm.at[pl.ds(off, cpairs)], irlo)
            pltpu.sync_copy(s_hbm.at[pl.ds(half + off, cpairs)], ishi)
            pltpu.sync_copy(r_hbm.at[pl.ds(half + off, cpairs)], irhi)
            c1 = pltpu.async_copy(ps_hbm.at[islo], bslo, sem)
            c2 = pltpu.async_copy(pr_hbm.at[irlo], brlo, sem)
            c3 = pltpu.async_copy(ps_hbm.at[ishi], bshi, sem)
            c4 = pltpu.async_copy(pr_hbm.at[irhi], brhi, sem)
            c1.wait()
            c2.wait()
            c3.wait()
            c4.wait()

            def add_row(rr, inner):
                for c4_ in range(HID // 16):
                    sl = pl.ds(c4_ * 16, 16)
                    sh = pl.ds(HID + c4_ * 16, 16)
                    bout[rr, sl] = bslo[rr, sl] + brlo[rr, sl]
                    bout[rr, sh] = bshi[rr, sl] + brhi[rr, sl]
                return inner

            lax.fori_loop(0, cpairs, add_row, 0)
            pltpu.sync_copy(bout, out_hbm.at[pl.ds(off, cpairs)])
            return carry

        lax.fori_loop(0, nchunk, chunk_body, 0)

    return _gather


def kernel(node_features, edge_index, edge_features, W1, b1, W2, b2, W3, b3):
    n_nodes = node_features.shape[0]
    n_edges = edge_features.shape[0]
    half = n_edges // 2

    s32 = edge_index[0].astype(jnp.int32)
    r32 = edge_index[1].astype(jnp.int32)
    w1st = W1[:, :NODE_D].T
    w1rt = W1[:, NODE_D:2 * NODE_D].T
    w1et = W1[:, 2 * NODE_D:].T
    b1_2 = b1.reshape(1, HID)
    b2_2 = b2.reshape(1, HID)
    b3_2 = b3.reshape(1, 1)

    ps, pr = pl.pallas_call(
        _precompute_body,
        out_shape=(
            jax.ShapeDtypeStruct((n_nodes, HID), jnp.float32),
            jax.ShapeDtypeStruct((n_nodes, HID), jnp.float32),
        ),
    )(node_features, w1st, w1rt)

    g2 = _make_gather(n_edges, 200)(ps, pr, s32, r32)

    rows = 1600
    nblocks = half // rows
    out_lo, out_hi = pl.pallas_call(
        _mlp_body,
        grid=(nblocks,),
        in_specs=[
            pl.BlockSpec((rows, 2 * HID), lambda i: (i, 0)),
            pl.BlockSpec((rows, EDGE_D), lambda i: (i, 0)),
            pl.BlockSpec((rows, EDGE_D), lambda i, _n=nblocks: (i + _n, 0)),
            pl.BlockSpec((EDGE_D, HID), lambda i: (0, 0)),
            pl.BlockSpec((1, HID), lambda i: (0, 0)),
            pl.BlockSpec((HID, HID), lambda i: (0, 0)),
            pl.BlockSpec((1, HID), lambda i: (0, 0)),
            pl.BlockSpec((1, HID), lambda i: (0, 0)),
            pl.BlockSpec((1, 1), lambda i: (0, 0)),
        ],
        out_specs=(
            pl.BlockSpec((1, 1, rows), lambda i: (i, 0, 0)),
            pl.BlockSpec((1, 1, rows), lambda i: (i, 0, 0)),
        ),
        out_shape=(
            jax.ShapeDtypeStruct((nblocks, 1, rows), jnp.float32),
            jax.ShapeDtypeStruct((nblocks, 1, rows), jnp.float32),
        ),
    )(g2, edge_features, edge_features, w1et, b1_2, W2.T, b2_2, W3, b3_2)
    return jnp.concatenate([out_lo.reshape(half), out_hi.reshape(half)])


# speedup vs baseline: 1.9366x; 1.9366x over previous
import functools

import jax
import jax.numpy as jnp
from jax import lax
from jax.experimental import pallas as pl
from jax.experimental.pallas import tpu as pltpu
from jax.experimental.pallas import tpu_sc as plsc

NODE_D = 128
HID = 64
EDGE_D = 16

_NC = 2
_NS = 16
_NW = _NC * _NS


def _elu(x):
    return jnp.where(x > 0, x, jnp.exp(x) - 1.0)


def _softplus(x):
    return jnp.maximum(x, 0.0) + jnp.log(1.0 + jnp.exp(-jnp.abs(x)))


def _precompute_body(nf_ref, wst_ref, wrt_ref, ps_ref, pr_ref):
    nf = nf_ref[...]
    ps_ref[...] = jnp.dot(nf, wst_ref[...], preferred_element_type=jnp.float32)
    pr_ref[...] = jnp.dot(nf, wrt_ref[...], preferred_element_type=jnp.float32)


def _mlp_body(g2_ref, eftlo_ref, efthi_ref, w1et_ref, b1_ref, w2t_ref, b2_ref,
              w3_ref, b3_ref, outlo_ref, outhi_ref):
    g2 = g2_ref[...]
    x = jnp.concatenate([g2[:, :HID], g2[:, HID:]], axis=0)
    eft = jnp.concatenate([eftlo_ref[...], efthi_ref[...]], axis=1)
    a = lax.dot_general(eft, w1et_ref[...], (((0,), (0,)), ((), ())),
                        preferred_element_type=jnp.float32)
    h1 = _elu(x + a + b1_ref[...])
    h2 = _elu(jnp.dot(h1, w2t_ref[...], preferred_element_type=jnp.float32)
              + b2_ref[...])
    z = lax.dot_general(w3_ref[...], h2, (((1,), (1,)), ((), ())),
                        preferred_element_type=jnp.float32)
    sp = _softplus(z + b3_ref[...])
    half = sp.shape[1] // 2
    outlo_ref[0] = sp[:, :half]
    outhi_ref[0] = sp[:, half:]


def _make_gather(n_edges, cpairs):
    half = n_edges // 2
    ppw = half // _NW
    nchunk = ppw // cpairs
    mesh = plsc.VectorSubcoreMesh(core_axis_name="c", subcore_axis_name="s")

    @functools.partial(
        pl.kernel,
        mesh=mesh,
        compiler_params=pltpu.CompilerParams(use_tc_tiling_on_sc=False),
        out_type=jax.ShapeDtypeStruct((half, 2 * HID), jnp.float32),
        scratch_types=[
            pltpu.VMEM((2 * ppw,), jnp.int32),
            pltpu.VMEM((2 * ppw,), jnp.int32),
            pltpu.VMEM((cpairs, HID), jnp.float32),
            pltpu.VMEM((cpairs, HID), jnp.float32),
            pltpu.VMEM((cpairs, HID), jnp.float32),
            pltpu.VMEM((cpairs, HID), jnp.float32),
            pltpu.VMEM((cpairs, HID), jnp.float32),
            pltpu.VMEM((cpairs, HID), jnp.float32),
            pltpu.SemaphoreType.DMA,
            pltpu.SemaphoreType.DMA,
            pltpu.SemaphoreType.DMA,
            pltpu.SemaphoreType.DMA,
        ],
    )
    def _gather(ps_hbm, pr_hbm, s_hbm, r_hbm, out_hbm, idx_s, idx_r,
                bs0, br0, bw0, bs1, br1, bw1, semg0, semg1, semw0, semw1):
        wid = lax.axis_index("s") * _NC + lax.axis_index("c")
        base = wid * ppw
        bufs = ((bs0, br0, bw0, semg0, semw0), (bs1, br1, bw1, semg1, semw1))

        pltpu.sync_copy(s_hbm.at[pl.ds(base, ppw)], idx_s.at[pl.ds(0, ppw)])
        pltpu.sync_copy(s_hbm.at[pl.ds(half + base, ppw)],
                        idx_s.at[pl.ds(ppw, ppw)])
        pltpu.sync_copy(r_hbm.at[pl.ds(base, ppw)], idx_r.at[pl.ds(0, ppw)])
        pltpu.sync_copy(r_hbm.at[pl.ds(half + base, ppw)],
                        idx_r.at[pl.ds(ppw, ppw)])

        def issue_gathers(b, k):
            bs, br, _, semg, _ = bufs[b]
            ioff = b * ppw + k * cpairs
            cs = pltpu.async_copy(
                ps_hbm.at[idx_s.at[pl.ds(ioff, cpairs)]], bs, semg)
            cr = pltpu.async_copy(
                pr_hbm.at[idx_r.at[pl.ds(ioff, cpairs)]], br, semg)
            return cs, cr

        issue_gathers(0, 0)
        issue_gathers(1, 0)

        def chunk_body(k, carry):
            for b in (0, 1):
                bs, br, bw, semg, semw = bufs[b]
                coff = b * HID
                row = base + k * cpairs

                pltpu.make_async_copy(
                    ps_hbm.at[idx_s.at[pl.ds(0, cpairs)]], bs, semg).wait()
                pltpu.make_async_copy(
                    pr_hbm.at[idx_r.at[pl.ds(0, cpairs)]], br, semg).wait()

                @pl.when(k >= 1)
                def _():
                    pltpu.make_async_copy(
                        bw, out_hbm.at[pl.ds(row - cpairs, cpairs),
                                       pl.ds(coff, HID)], semw).wait()

                def add_row(rr, inner):
                    for c4 in range(HID // 16):
                        sl = pl.ds(c4 * 16, 16)
                        bw[rr, sl] = bs[rr, sl] + br[rr, sl]
                    return inner

                lax.fori_loop(0, cpairs, add_row, 0)

                @pl.when(k < nchunk - 1)
                def _():
                    issue_gathers(b, k + 1)

                pltpu.async_copy(
                    bw, out_hbm.at[pl.ds(row, cpairs), pl.ds(coff, HID)], semw)
            return carry

        lax.fori_loop(0, nchunk, chunk_body, 0)

        for b in (0, 1):
            _, _, bw, _, semw = bufs[b]
            row = base + (nchunk - 1) * cpairs
            pltpu.make_async_copy(
                bw, out_hbm.at[pl.ds(row, cpairs), pl.ds(b * HID, HID)],
                semw).wait()

    return _gather


def kernel(node_features, edge_index, edge_features, W1, b1, W2, b2, W3, b3):
    n_nodes = node_features.shape[0]
    n_edges = edge_features.shape[0]
    half = n_edges // 2

    s32 = edge_index[0].astype(jnp.int32)
    r32 = edge_index[1].astype(jnp.int32)
    eft = edge_features.T
    w1st = W1[:, :NODE_D].T
    w1rt = W1[:, NODE_D:2 * NODE_D].T
    w1et = W1[:, 2 * NODE_D:].T
    b1_2 = b1.reshape(1, HID)
    b2_2 = b2.reshape(1, HID)
    b3_2 = b3.reshape(1, 1)

    ps, pr = pl.pallas_call(
        _precompute_body,
        out_shape=(
            jax.ShapeDtypeStruct((n_nodes, HID), jnp.float32),
            jax.ShapeDtypeStruct((n_nodes, HID), jnp.float32),
        ),
    )(node_features, w1st, w1rt)

    g2 = _make_gather(n_edges, 200)(ps, pr, s32, r32)

    rows = 1280
    nblocks = half // rows
    out_lo, out_hi = pl.pallas_call(
        _mlp_body,
        grid=(nblocks,),
        in_specs=[
            pl.BlockSpec((rows, 2 * HID), lambda i: (i, 0)),
            pl.BlockSpec((EDGE_D, rows), lambda i: (0, i)),
            pl.BlockSpec((EDGE_D, rows), lambda i, _n=nblocks: (0, i + _n)),
            pl.BlockSpec((EDGE_D, HID), lambda i: (0, 0)),
            pl.BlockSpec((1, HID), lambda i: (0, 0)),
            pl.BlockSpec((HID, HID), lambda i: (0, 0)),
            pl.BlockSpec((1, HID), lambda i: (0, 0)),
            pl.BlockSpec((1, HID), lambda i: (0, 0)),
            pl.BlockSpec((1, 1), lambda i: (0, 0)),
        ],
        out_specs=(
            pl.BlockSpec((1, 1, rows), lambda i: (i, 0, 0)),
            pl.BlockSpec((1, 1, rows), lambda i: (i, 0, 0)),
        ),
        out_shape=(
            jax.ShapeDtypeStruct((nblocks, 1, rows), jnp.float32),
            jax.ShapeDtypeStruct((nblocks, 1, rows), jnp.float32),
        ),
    )(g2, eft, eft, w1et, b1_2, W2.T, b2_2, W3, b3_2)
    return jnp.concatenate([out_lo.reshape(half), out_hi.reshape(half)])
